# Initial kernel scaffold; baseline (speedup 1.0000x reference)
import jax
import jax.numpy as jnp
from jax.experimental import pallas as pl


def kernel(x, dim, index, value):
    raise NotImplementedError("write your pallas kernel here")


# speedup vs baseline: 9.0266x; 9.0266x over previous
import functools

import jax
import jax.numpy as jnp
from jax import lax
from jax.experimental import pallas as pl
from jax.experimental.pallas import tpu as pltpu
from jax.experimental.pallas import tpu_sc as plsc

M, D, B = 100000, 64, 16384
NIDX = B * D
NELEM = M * D
NSC = 2
NTILE = 16
CHUNKS_PER_SC = 2
CHUNK_ROWS = M // (NSC * CHUNKS_PER_SC)
CHUNK = CHUNK_ROWS * D
PAD = 2048
SLAB = NIDX // NTILE
TSLICE = CHUNK // NTILE
QSLICE = TSLICE // 20
BATCH = 8192
NBATCH = SLAB // BATCH

_mesh = plsc.VectorSubcoreMesh(core_axis_name="c", subcore_axis_name="s")


@functools.partial(
    pl.kernel,
    out_type=jax.ShapeDtypeStruct((NELEM,), jnp.float32),
    mesh=_mesh,
    scratch_types=[
        pltpu.VMEM_SHARED((CHUNK + PAD,), jnp.float32),
        pltpu.VMEM((BATCH,), jnp.int32),
        pltpu.VMEM((BATCH,), jnp.int32),
        pltpu.VMEM((BATCH,), jnp.float32),
        pltpu.VMEM((QSLICE,), jnp.float32),
    ],
)
def _scatter_add(x_hbm, idx_hbm, val_hbm, out_hbm, acc, ibuf, fbuf, vals, bounce):
    c = lax.axis_index("c")
    s = lax.axis_index("s")
    iota = lax.iota(jnp.int32, 16)

    pltpu.sync_copy(val_hbm, vals)

    for kk in range(CHUNKS_PER_SC):
        ebase = (c * CHUNKS_PER_SC + kk) * CHUNK
        coladj = [iota + (j * 16 - ebase) for j in range(4)]
        dummy = [iota + (CHUNK + 128 * s + 16 * j) for j in range(4)]

        for q in range(TSLICE // QSLICE):
            off = ebase + s * TSLICE + q * QSLICE
            pltpu.sync_copy(x_hbm.at[pl.ds(off, QSLICE)], bounce)
            pltpu.sync_copy(bounce, acc.at[pl.ds(s * TSLICE + q * QSLICE, QSLICE)])
        plsc.subcore_barrier()

        for b in range(NBATCH):
            pltpu.sync_copy(idx_hbm.at[pl.ds(s * SLAB + b * BATCH, BATCH)], ibuf)

            @plsc.parallel_loop(0, BATCH // 64, unroll=2)
            def _(t):
                for j in range(4):
                    v = ibuf[pl.ds(t * 64 + j * 16, 16)]
                    l = v * 64 + coladj[j]
                    ok = plsc.bitcast(l, jnp.uint32) < jnp.uint32(CHUNK)
                    fbuf[pl.ds(t * 64 + j * 16, 16)] = jnp.where(ok, l, dummy[j])

            pltpu.sync_copy(vals, acc.at[fbuf], add=True)

        plsc.subcore_barrier()
        for q in range(TSLICE // QSLICE):
            off = ebase + s * TSLICE + q * QSLICE
            pltpu.sync_copy(acc.at[pl.ds(s * TSLICE + q * QSLICE, QSLICE)], bounce)
            pltpu.sync_copy(bounce, out_hbm.at[pl.ds(off, QSLICE)])


def kernel(x, dim, index, value):
    row = index + jnp.asarray(dim, dtype=index.dtype)
    idx_flat = row.astype(jnp.int32).reshape(-1)
    vals = jnp.full((BATCH,), value, dtype=jnp.float32)
    out = _scatter_add(x.reshape(-1), idx_flat, vals)
    return out.reshape(M, D)
